# Initial kernel scaffold; baseline (speedup 1.0000x reference)
import jax
import jax.numpy as jnp
from jax.experimental import pallas as pl


def kernel(x, conv1_w, conv1_b, conv2_w, conv2_b, gcn1_w, gcn1_b, gcn2_w, gcn2_b, fc_w, fc_b):
    raise NotImplementedError("write your pallas kernel here")


# speedup vs baseline: 121.3755x; 121.3755x over previous
import jax
import jax.numpy as jnp
from jax.experimental import pallas as pl
from jax.experimental.pallas import tpu as pltpu

B = 256
C_IN = 32
T0 = 2048
U = 512
HIDDEN = 64
OUT = 32
KNN = 8


def _mm(a, w):
    return jax.lax.dot_general(a, w, (((1,), (0,)), ((), ())),
                               preferred_element_type=jnp.float32)


def _conv_body(xph_ref, w1_ref, b1_ref, w2_ref, b2_ref, out_ref):
    ph = [xph_ref[0, j] for j in range(4)]
    iota = jax.lax.broadcasted_iota(jnp.int32, (U, 1), 0)

    def up(a):
        return jnp.where(iota == 0, 0.0, jnp.roll(a, 1, axis=0))

    def dn(a):
        return jnp.where(iota == U - 1, 0.0, jnp.roll(a, -1, axis=0))

    w1 = [w1_ref[dt] for dt in range(5)]
    b1 = b1_ref[0]
    out1 = []
    for j in range(4):
        acc = None
        for dt in range(5):
            s = j + dt - 2
            if s < 0:
                a = up(ph[s + 4])
            elif s >= 4:
                a = dn(ph[s - 4])
            else:
                a = ph[s]
            t = _mm(a, w1[dt])
            acc = t if acc is None else acc + t
        out1.append(jnp.maximum(acc + b1, 0.0))
    p1 = [jnp.maximum(out1[0], out1[1]), jnp.maximum(out1[2], out1[3])]

    w2 = [w2_ref[dt] for dt in range(5)]
    b2 = b2_ref[0]
    seq0 = [up(p1[0]), up(p1[1]), p1[0], p1[1], dn(p1[0])]
    seq1 = [up(p1[1]), p1[0], p1[1], dn(p1[0]), dn(p1[1])]
    o20 = b2
    o21 = b2
    for dt in range(5):
        o20 = o20 + _mm(seq0[dt], w2[dt])
        o21 = o21 + _mm(seq1[dt], w2[dt])
    out_ref[0] = jnp.maximum(jnp.maximum(o20, 0.0), jnp.maximum(o21, 0.0))


def _graph_body(h0_ref, m_ref):
    h0 = h0_ref[...]
    g = jax.lax.dot_general(h0, h0, (((1,), (1,)), ((), ())),
                            preferred_element_type=jnp.float32)
    sqc = jnp.sum(h0 * h0, axis=1, keepdims=True)
    sqr = jnp.sum(h0 * h0, axis=1)[None, :]
    row = jax.lax.broadcasted_iota(jnp.int32, (U, U), 0)
    col = jax.lax.broadcasted_iota(jnp.int32, (U, U), 1)
    eye = row == col
    d2 = sqc + sqr - 2.0 * g + jnp.where(eye, 1e9, 0.0)
    acc = jnp.where(eye, 1.0, 0.0)
    for _ in range(KNN):
        m = jnp.min(d2, axis=1, keepdims=True)
        cand = jnp.where(d2 == m, col, U)
        idx = jnp.min(cand, axis=1, keepdims=True)
        sel = col == idx
        acc = acc + jnp.where(sel, 1.0, 0.0)
        d2 = jnp.where(sel, 3e9, d2)
    m_ref[...] = acc * (1.0 / 9.0)


GB = 8


def _gcn_body(m_ref, h_ref, w1_ref, b1_ref, w2_ref, b2_ref,
              fcw_ref, fcb_ref, out_ref):
    mop = m_ref[...]
    pooled = []
    for i in range(GB):
        h = h_ref[i]
        a1 = jnp.maximum(_mm(mop, _mm(h, w1_ref[...])) + b1_ref[0], 0.0)
        a2 = jnp.maximum(_mm(mop, _mm(a1, w2_ref[...])) + b2_ref[0], 0.0)
        pooled.append(jnp.sum(a2, axis=0, keepdims=True) * (1.0 / U))
    pooled = jnp.concatenate(pooled, axis=0)
    out_ref[...] = _mm(pooled, fcw_ref[...]) + fcb_ref[0]


def kernel(x, conv1_w, conv1_b, conv2_w, conv2_b,
           gcn1_w, gcn1_b, gcn2_w, gcn2_b, fc_w, fc_b):
    xph = jnp.transpose(x, (0, 2, 1)).reshape(B, U, 4, C_IN)
    xph = jnp.transpose(xph, (0, 2, 1, 3))
    w1t = jnp.transpose(conv1_w, (2, 1, 0))
    w2t = jnp.transpose(conv2_w, (2, 1, 0))
    fcwt = jnp.transpose(fc_w, (1, 0))
    b1 = conv1_b[None, :]
    b2 = conv2_b[None, :]
    g1b = gcn1_b[None, :]
    g2b = gcn2_b[None, :]
    fcb = fc_b[None, :]

    h = pl.pallas_call(
        _conv_body,
        grid=(B,),
        in_specs=[
            pl.BlockSpec((1, 4, U, C_IN), lambda b: (b, 0, 0, 0)),
            pl.BlockSpec((5, C_IN, 16), lambda b: (0, 0, 0)),
            pl.BlockSpec((1, 16), lambda b: (0, 0)),
            pl.BlockSpec((5, 16, C_IN), lambda b: (0, 0, 0)),
            pl.BlockSpec((1, C_IN), lambda b: (0, 0)),
        ],
        out_specs=pl.BlockSpec((1, U, C_IN), lambda b: (b, 0, 0)),
        out_shape=jax.ShapeDtypeStruct((B, U, C_IN), jnp.float32),
    )(xph, w1t, b1, w2t, b2)

    mop = pl.pallas_call(
        _graph_body,
        out_shape=jax.ShapeDtypeStruct((U, U), jnp.float32),
    )(h[0])

    out = pl.pallas_call(
        _gcn_body,
        grid=(B // GB,),
        in_specs=[
            pl.BlockSpec((U, U), lambda b: (0, 0)),
            pl.BlockSpec((GB, U, C_IN), lambda b: (b, 0, 0)),
            pl.BlockSpec((C_IN, HIDDEN), lambda b: (0, 0)),
            pl.BlockSpec((1, HIDDEN), lambda b: (0, 0)),
            pl.BlockSpec((HIDDEN, HIDDEN), lambda b: (0, 0)),
            pl.BlockSpec((1, HIDDEN), lambda b: (0, 0)),
            pl.BlockSpec((HIDDEN, OUT), lambda b: (0, 0)),
            pl.BlockSpec((1, OUT), lambda b: (0, 0)),
        ],
        out_specs=pl.BlockSpec((GB, OUT), lambda b: (b, 0)),
        out_shape=jax.ShapeDtypeStruct((B, OUT), jnp.float32),
    )(mop, h, gcn1_w, g1b, gcn2_w, g2b, fcwt, fcb)
    return out
